# Initial kernel scaffold; baseline (speedup 1.0000x reference)
import jax
import jax.numpy as jnp
from jax.experimental import pallas as pl


def kernel(z, edge_index, edge_weight):
    raise NotImplementedError("write your pallas kernel here")


# speedup vs baseline: 1.8886x; 1.8886x over previous
import jax
import jax.numpy as jnp
from jax import lax
from jax.experimental import pallas as pl
from jax.experimental.pallas import tpu as pltpu
from jax.experimental.pallas import tpu_sc as plsc

N_NODES = 10000
N_FEAT = 256
NUM_TILES = 32
COLS = N_FEAT // NUM_TILES
N_EDGES = 160000
CHUNK = 4000


def _sc_body(zt_hbm, row_hbm, col_hbm, w_hbm, out_hbm,
             ztile, r_v, c_v, w_v, acc_v):
    wid = lax.axis_index("s") * 2 + lax.axis_index("c")
    pltpu.sync_copy(zt_hbm.at[wid], ztile)

    def chunk_body(ci, acc):
        off = ci * CHUNK
        pltpu.sync_copy(row_hbm.at[pl.ds(off, CHUNK)], r_v)
        pltpu.sync_copy(col_hbm.at[pl.ds(off, CHUNK)], c_v)
        pltpu.sync_copy(w_hbm.at[pl.ds(off, CHUNK)], w_v)

        def group_body(g, acc):
            r16 = r_v[pl.ds(g * 16, 16)]
            c16 = c_v[pl.ds(g * 16, 16)]
            w16 = w_v[pl.ds(g * 16, 16)]
            w2 = w16 * w16
            ir = r16 * COLS
            ic = c16 * COLS
            for j in range(COLS):
                a = plsc.load_gather(ztile, [ir + j])
                b = plsc.load_gather(ztile, [ic + j])
                d = a - b
                acc = acc + (w2 * d) * d
            return acc

        return lax.fori_loop(0, CHUNK // 16, group_body, acc)

    acc = lax.fori_loop(0, N_EDGES // CHUNK, chunk_body,
                        jnp.zeros((16,), jnp.float32))
    acc_v[...] = acc
    pltpu.sync_copy(acc_v, out_hbm.at[wid])


_sc_call = pl.kernel(
    _sc_body,
    out_type=jax.ShapeDtypeStruct((NUM_TILES, 16), jnp.float32),
    mesh=plsc.VectorSubcoreMesh(core_axis_name="c", subcore_axis_name="s"),
    scratch_types=[
        pltpu.VMEM((N_NODES * COLS,), jnp.float32),
        pltpu.VMEM((CHUNK,), jnp.int32),
        pltpu.VMEM((CHUNK,), jnp.int32),
        pltpu.VMEM((CHUNK,), jnp.float32),
        pltpu.VMEM((16,), jnp.float32),
    ],
    compiler_params=pltpu.CompilerParams(needs_layout_passes=False),
)


def kernel(z, edge_index, edge_weight):
    row = edge_index[0].astype(jnp.int32)
    col = edge_index[1].astype(jnp.int32)
    zt = z.reshape(N_NODES, NUM_TILES, COLS).transpose(1, 0, 2)
    zt = zt.reshape(NUM_TILES, N_NODES * COLS)
    partials = _sc_call(zt, row, col, edge_weight)
    return jnp.sum(partials) / edge_index.shape[1]


# speedup vs baseline: 3.7726x; 1.9976x over previous
import jax
import jax.numpy as jnp
from jax import lax
from jax.experimental import pallas as pl
from jax.experimental.pallas import tpu as pltpu
from jax.experimental.pallas import tpu_sc as plsc

N_NODES = 10000
N_FEAT = 256
NFG = 16
WPN = N_FEAT // NFG // 2
NEG = 2
N_EDGES = 160000
EDGES_PER_EG = N_EDGES // NEG
CHUNK = 8000
NCHUNK = EDGES_PER_EG // CHUNK
GROUPS = CHUNK // 16


def _sc_body(zt_hbm, row_hbm, col_hbm, w_hbm, out_hbm,
             zt0, zt1, zt2, zt3, zt4, zt5, zt6, zt7,
             r0, c0, w0, r1, c1, w1, acc_v, sem0, sem1):
    ztiles = (zt0, zt1, zt2, zt3, zt4, zt5, zt6, zt7)
    wid = lax.axis_index("s") * 2 + lax.axis_index("c")
    fg = wid % NFG
    eg = wid // NFG
    for j in range(WPN):
        pltpu.sync_copy(zt_hbm.at[fg, j], ztiles[j])
    ebase = eg * EDGES_PER_EG

    bufs = ((r0, c0, w0, sem0), (r1, c1, w1, sem1))

    def start_chunk(ci):
        off = ebase + ci * CHUNK
        r, c, w, sem = bufs[ci % 2]
        return (pltpu.async_copy(row_hbm.at[pl.ds(off, CHUNK)], r, sem),
                pltpu.async_copy(col_hbm.at[pl.ds(off, CHUNK)], c, sem),
                pltpu.async_copy(w_hbm.at[pl.ds(off, CHUNK)], w, sem))

    def run_chunk(ci, accs):
        r_v, c_v, w_v, _ = bufs[ci % 2]

        @plsc.parallel_loop(0, GROUPS, carry=accs)
        def accs_out(g, acc):
            r16 = r_v[pl.ds(g * 16, 16)]
            c16 = c_v[pl.ds(g * 16, 16)]
            w16 = w_v[pl.ds(g * 16, 16)]
            w2 = w16 * w16
            new = []
            for j in range(WPN):
                gr = plsc.load_gather(ztiles[j], [r16])
                gc = plsc.load_gather(ztiles[j], [c16])
                d = (plsc.bitcast(gr, jnp.bfloat16)
                     - plsc.bitcast(gc, jnp.bfloat16))
                di = plsc.bitcast(d, jnp.int32)
                dlo = plsc.bitcast(di << 16, jnp.float32)
                dhi = plsc.bitcast(di & jnp.int32(-65536), jnp.float32)
                new.append(acc[2 * j] + (dlo * dlo) * w2)
                new.append(acc[2 * j + 1] + (dhi * dhi) * w2)
            return tuple(new)

        return accs_out

    accs = tuple(jnp.zeros((16,), jnp.float32) for _ in range(2 * WPN))
    descs = start_chunk(0)
    for ci in range(NCHUNK):
        nxt = start_chunk(ci + 1) if ci + 1 < NCHUNK else ()
        for dsc in descs:
            dsc.wait()
        accs = run_chunk(ci, accs)
        descs = nxt

    total = accs[0]
    for a in accs[1:]:
        total = total + a
    acc_v[...] = total
    pltpu.sync_copy(acc_v, out_hbm.at[wid])


_sc_call = pl.kernel(
    _sc_body,
    out_type=jax.ShapeDtypeStruct((NEG * NFG, 16), jnp.float32),
    mesh=plsc.VectorSubcoreMesh(core_axis_name="c", subcore_axis_name="s"),
    scratch_types=[
        pltpu.VMEM((N_NODES,), jnp.int32),
        pltpu.VMEM((N_NODES,), jnp.int32),
        pltpu.VMEM((N_NODES,), jnp.int32),
        pltpu.VMEM((N_NODES,), jnp.int32),
        pltpu.VMEM((N_NODES,), jnp.int32),
        pltpu.VMEM((N_NODES,), jnp.int32),
        pltpu.VMEM((N_NODES,), jnp.int32),
        pltpu.VMEM((N_NODES,), jnp.int32),
        pltpu.VMEM((CHUNK,), jnp.int32),
        pltpu.VMEM((CHUNK,), jnp.int32),
        pltpu.VMEM((CHUNK,), jnp.float32),
        pltpu.VMEM((CHUNK,), jnp.int32),
        pltpu.VMEM((CHUNK,), jnp.int32),
        pltpu.VMEM((CHUNK,), jnp.float32),
        pltpu.VMEM((16,), jnp.float32),
        pltpu.SemaphoreType.DMA,
        pltpu.SemaphoreType.DMA,
    ],
    compiler_params=pltpu.CompilerParams(needs_layout_passes=False),
)


def kernel(z, edge_index, edge_weight):
    row = edge_index[0].astype(jnp.int32)
    col = edge_index[1].astype(jnp.int32)
    zbf = z.astype(jnp.bfloat16).reshape(N_NODES, NFG, WPN, 2)
    zt = jax.lax.bitcast_convert_type(zbf, jnp.int32)
    zt = zt.transpose(1, 2, 0)
    partials = _sc_call(zt, row, col, edge_weight)
    return jnp.sum(partials) / edge_index.shape[1]


# speedup vs baseline: 4.1697x; 1.1053x over previous
import jax
import jax.numpy as jnp
from jax import lax
from jax.experimental import pallas as pl
from jax.experimental.pallas import tpu as pltpu
from jax.experimental.pallas import tpu_sc as plsc

N_NODES = 10000
N_FEAT = 256
NFG = 16
WPN = N_FEAT // NFG // 2
NEG = 2
N_EDGES = 160000
EDGES_PER_EG = N_EDGES // NEG
CHUNK = 8000
NCHUNK = EDGES_PER_EG // CHUNK
GROUPS = CHUNK // 16


def _sc_body(zt_hbm, row_hbm, col_hbm, w_hbm, out_hbm,
             zt0, zt1, zt2, zt3, zt4, zt5, zt6, zt7,
             r0, c0, w0, r1, c1, w1, acc_v, sem0, sem1):
    ztiles = (zt0, zt1, zt2, zt3, zt4, zt5, zt6, zt7)
    wid = lax.axis_index("s") * 2 + lax.axis_index("c")
    fg = wid % NFG
    eg = wid // NFG
    for j in range(WPN):
        pltpu.sync_copy(zt_hbm.at[fg * WPN + j], ztiles[j])
    ebase = eg * EDGES_PER_EG

    bufs = ((r0, c0, w0, sem0), (r1, c1, w1, sem1))

    def start_chunk(ci):
        off = ebase + ci * CHUNK
        r, c, w, sem = bufs[ci % 2]
        return (pltpu.async_copy(row_hbm.at[pl.ds(off, CHUNK)], r, sem),
                pltpu.async_copy(col_hbm.at[pl.ds(off, CHUNK)], c, sem),
                pltpu.async_copy(w_hbm.at[pl.ds(off, CHUNK)], w, sem))

    def run_chunk(ci, acc0):
        r_v, c_v, w_v, _ = bufs[ci % 2]

        @plsc.parallel_loop(0, GROUPS, carry=acc0)
        def acc_out(g, acc):
            r16 = r_v[pl.ds(g * 16, 16)]
            c16 = c_v[pl.ds(g * 16, 16)]
            w16 = w_v[pl.ds(g * 16, 16)]
            w2 = w16 * w16
            slo = None
            shi = None
            for j in range(WPN):
                gr = plsc.load_gather(ztiles[j], [r16])
                gc = plsc.load_gather(ztiles[j], [c16])
                d = (plsc.bitcast(gr, jnp.bfloat16)
                     - plsc.bitcast(gc, jnp.bfloat16))
                d2 = plsc.bitcast(d * d, jnp.int32)
                d2lo = plsc.bitcast(d2 << 16, jnp.float32)
                d2hi = plsc.bitcast(d2 & jnp.int32(-65536), jnp.float32)
                slo = d2lo if slo is None else slo + d2lo
                shi = d2hi if shi is None else shi + d2hi
            return acc + w2 * (slo + shi)

        return acc_out

    acc = jnp.zeros((16,), jnp.float32)
    descs = start_chunk(0)
    for ci in range(NCHUNK):
        nxt = start_chunk(ci + 1) if ci + 1 < NCHUNK else ()
        for dsc in descs:
            dsc.wait()
        acc = run_chunk(ci, acc)
        descs = nxt

    acc_v[...] = acc
    pltpu.sync_copy(acc_v, out_hbm.at[wid])


_sc_call = pl.kernel(
    _sc_body,
    out_type=jax.ShapeDtypeStruct((NEG * NFG, 16), jnp.float32),
    mesh=plsc.VectorSubcoreMesh(core_axis_name="c", subcore_axis_name="s"),
    scratch_types=[
        pltpu.VMEM((N_NODES,), jnp.int32),
        pltpu.VMEM((N_NODES,), jnp.int32),
        pltpu.VMEM((N_NODES,), jnp.int32),
        pltpu.VMEM((N_NODES,), jnp.int32),
        pltpu.VMEM((N_NODES,), jnp.int32),
        pltpu.VMEM((N_NODES,), jnp.int32),
        pltpu.VMEM((N_NODES,), jnp.int32),
        pltpu.VMEM((N_NODES,), jnp.int32),
        pltpu.VMEM((CHUNK,), jnp.int32),
        pltpu.VMEM((CHUNK,), jnp.int32),
        pltpu.VMEM((CHUNK,), jnp.float32),
        pltpu.VMEM((CHUNK,), jnp.int32),
        pltpu.VMEM((CHUNK,), jnp.int32),
        pltpu.VMEM((CHUNK,), jnp.float32),
        pltpu.VMEM((16,), jnp.float32),
        pltpu.SemaphoreType.DMA,
        pltpu.SemaphoreType.DMA,
    ],
    compiler_params=pltpu.CompilerParams(needs_layout_passes=False),
)


def kernel(z, edge_index, edge_weight):
    row = edge_index[0].astype(jnp.int32)
    col = edge_index[1].astype(jnp.int32)
    zbf = z.astype(jnp.bfloat16).reshape(N_NODES, N_FEAT // 2, 2)
    zp = jax.lax.bitcast_convert_type(zbf, jnp.int32)
    zt = zp.T
    partials = _sc_call(zt, row, col, edge_weight)
    return jnp.sum(partials) / edge_index.shape[1]
